# Initial kernel scaffold; baseline (speedup 1.0000x reference)
import jax
import jax.numpy as jnp
from jax.experimental import pallas as pl


def kernel(prior_features, feature_clip, norm1_w, norm1_b, norm2_w, norm2_b):
    raise NotImplementedError("write your pallas kernel here")


# speedup vs baseline: 2.6802x; 2.6802x over previous
import functools

import jax
import jax.numpy as jnp
from jax import lax
from jax.experimental import pallas as pl
from jax.experimental.pallas import tpu as pltpu
from jax.experimental.pallas import tpu_sc as plsc

DIM = 64
NKEY = 1_000_000
NQ = 16
KTOP = 64
BLK = 16384
SUB = 128
NSUB_B = BLK // SUB
NBLK = -(-NKEY // BLK)
NKEYP = NBLK * BLK
NSUB = NKEYP // SUB
NCAND = KTOP * SUB
EPS = 1e-5
NEG = float("-inf")
IBIG = 2**31 - 1


def _score_body(pf_ref, fc_ref, w1_ref, b1_ref, w2_ref, b2_ref,
                sim_ref, m_ref):
    b = pl.program_id(0)
    x = pf_ref[...]
    mu = jnp.mean(x, axis=-1, keepdims=True)
    var = jnp.var(x, axis=-1, keepdims=True)
    x1 = (x - mu) / jnp.sqrt(var + EPS) * w1_ref[...] + b1_ref[...]
    n1 = jnp.sqrt(jnp.sum(x1 * x1, axis=-1))

    q = fc_ref[...]
    qmu = jnp.mean(q, axis=-1, keepdims=True)
    qvar = jnp.var(q, axis=-1, keepdims=True)
    x2 = (q - qmu) / jnp.sqrt(qvar + EPS) * w2_ref[...] + b2_ref[...]
    n2 = jnp.sqrt(jnp.sum(x2 * x2, axis=-1, keepdims=True))

    dots = lax.dot_general(x2, x1, (((1,), (1,)), ((), ())),
                           preferred_element_type=jnp.float32)
    denom = jnp.maximum(n2 * n1.reshape(1, BLK), 1e-8)
    sim = dots / denom

    gk = b * BLK + lax.broadcasted_iota(jnp.int32, (1, BLK), 1)
    sim = jnp.where(gk < NKEY, sim, NEG)
    sim3 = sim.reshape(NQ, NSUB_B, SUB)
    sim_ref[...] = sim3
    m_ref[...] = jnp.max(sim3, axis=2)


def _subsel_body(m_ref, out_ref, v_ref):
    v_ref[...] = m_ref[...]
    sid = lax.broadcasted_iota(jnp.int32, (NQ, NSUB), 1)
    kio = lax.broadcasted_iota(jnp.int32, (NQ, KTOP), 1)
    qid = lax.broadcasted_iota(jnp.int32, (NQ, 1), 0)

    def body(r, outv):
        v = v_ref[...]
        m = jnp.max(v, axis=1, keepdims=True)
        sel = jnp.min(jnp.where(v == m, sid, IBIG), axis=1, keepdims=True)
        v_ref[...] = jnp.where(sid == sel, NEG, v)
        return jnp.where(kio == r, qid * NSUB + sel, outv)

    out_ref[...] = lax.fori_loop(0, KTOP, body, jnp.zeros((NQ, KTOP), jnp.int32))


def _finsel_body(cand_ref, gidx_ref, out_ref, v_ref):
    v_ref[...] = cand_ref[...]
    gidx = gidx_ref[...]
    kio = lax.broadcasted_iota(jnp.int32, (NQ, KTOP), 1)

    def body(r, outv):
        v = v_ref[...]
        m = jnp.max(v, axis=1, keepdims=True)
        sel = jnp.min(jnp.where(v == m, gidx, IBIG), axis=1, keepdims=True)
        v_ref[...] = jnp.where(gidx == sel, NEG, v)
        outf = sel.astype(jnp.float32) / float(NKEY)
        return jnp.where(kio == r, outf, outv)

    out_ref[...] = lax.fori_loop(0, KTOP, body,
                                 jnp.zeros((NQ, KTOP), jnp.float32))


def _sc_gather(sim_flat, rows_flat):
    nrows = NQ * KTOP
    info = plsc.get_sparse_core_info()
    nw = info.num_cores * info.num_subcores
    per_w = nrows // nw
    mesh = plsc.VectorSubcoreMesh(core_axis_name="c", subcore_axis_name="s")

    @functools.partial(
        pl.kernel, mesh=mesh,
        out_type=jax.ShapeDtypeStruct((nrows, SUB), jnp.float32),
        scratch_types=[
            pltpu.VMEM((per_w,), jnp.int32),
            pltpu.VMEM((per_w, SUB), jnp.float32),
            pltpu.SemaphoreType.DMA,
        ],
    )
    def gather_k(sim_hbm, rows_hbm, out_hbm, idx_v, rows_v, sem):
        wid = lax.axis_index("s") * info.num_cores + lax.axis_index("c")
        base = wid * per_w
        pltpu.sync_copy(rows_hbm.at[pl.ds(base, per_w)], idx_v)
        pltpu.async_copy(sim_hbm.at[idx_v], rows_v, sem).wait()
        pltpu.sync_copy(rows_v, out_hbm.at[pl.ds(base, per_w)])

    return gather_k(sim_flat, rows_flat)


def kernel(prior_features, feature_clip, norm1_w, norm1_b, norm2_w, norm2_b):
    pf = prior_features.reshape(NKEY, DIM)
    w1 = norm1_w.reshape(1, DIM)
    b1 = norm1_b.reshape(1, DIM)
    w2 = norm2_w.reshape(1, DIM)
    b2 = norm2_b.reshape(1, DIM)

    sim3, msub = pl.pallas_call(
        _score_body,
        grid=(NBLK,),
        in_specs=[
            pl.BlockSpec((BLK, DIM), lambda b: (b, 0)),
            pl.BlockSpec((NQ, DIM), lambda b: (0, 0)),
            pl.BlockSpec((1, DIM), lambda b: (0, 0)),
            pl.BlockSpec((1, DIM), lambda b: (0, 0)),
            pl.BlockSpec((1, DIM), lambda b: (0, 0)),
            pl.BlockSpec((1, DIM), lambda b: (0, 0)),
        ],
        out_specs=[
            pl.BlockSpec((NQ, NSUB_B, SUB), lambda b: (0, b, 0)),
            pl.BlockSpec((NQ, NSUB_B), lambda b: (0, b)),
        ],
        out_shape=[
            jax.ShapeDtypeStruct((NQ, NSUB, SUB), jnp.float32),
            jax.ShapeDtypeStruct((NQ, NSUB), jnp.float32),
        ],
    )(pf, feature_clip, w1, b1, w2, b2)

    rows2 = pl.pallas_call(
        _subsel_body,
        out_shape=jax.ShapeDtypeStruct((NQ, KTOP), jnp.int32),
        scratch_shapes=[pltpu.VMEM((NQ, NSUB), jnp.float32)],
    )(msub)

    cand = _sc_gather(sim3.reshape(NQ * NSUB, SUB), rows2.reshape(-1))

    bid = rows2 - jnp.arange(NQ, dtype=jnp.int32)[:, None] * NSUB
    gidx = (bid[:, :, None] * SUB
            + jnp.arange(SUB, dtype=jnp.int32)).reshape(NQ, NCAND)

    out = pl.pallas_call(
        _finsel_body,
        out_shape=jax.ShapeDtypeStruct((NQ, KTOP), jnp.float32),
        scratch_shapes=[pltpu.VMEM((NQ, NCAND), jnp.float32)],
    )(cand.reshape(NQ, NCAND), gidx)

    return out


# speedup vs baseline: 2.6808x; 1.0002x over previous
import functools

import jax
import jax.numpy as jnp
from jax import lax
from jax.experimental import pallas as pl
from jax.experimental.pallas import tpu as pltpu
from jax.experimental.pallas import tpu_sc as plsc

DIM = 64
NKEY = 1_000_000
NQ = 16
KTOP = 64
BLK = 16384
SUB = 128
NSUB_B = BLK // SUB
NBLK = -(-NKEY // BLK)
NKEYP = NBLK * BLK
NSUB = NKEYP // SUB
NCAND = KTOP * SUB
EPS = 1e-5
NEG = float("-inf")
IBIG = 2**31 - 1


def _score_body(pf_ref, fc_ref, w1_ref, b1_ref, w2_ref, b2_ref,
                sim_ref, m_ref):
    b = pl.program_id(0)
    x = pf_ref[...]
    mu = jnp.mean(x, axis=-1, keepdims=True)
    var = jnp.var(x, axis=-1, keepdims=True)
    x1 = (x - mu) / jnp.sqrt(var + EPS) * w1_ref[...] + b1_ref[...]
    n1 = jnp.sqrt(jnp.sum(x1 * x1, axis=-1))

    q = fc_ref[...]
    qmu = jnp.mean(q, axis=-1, keepdims=True)
    qvar = jnp.var(q, axis=-1, keepdims=True)
    x2 = (q - qmu) / jnp.sqrt(qvar + EPS) * w2_ref[...] + b2_ref[...]
    n2 = jnp.sqrt(jnp.sum(x2 * x2, axis=-1, keepdims=True))

    dots = lax.dot_general(x2, x1, (((1,), (1,)), ((), ())),
                           preferred_element_type=jnp.float32)
    denom = jnp.maximum(n2 * n1.reshape(1, BLK), 1e-8)
    sim = dots / denom

    gk = b * BLK + lax.broadcasted_iota(jnp.int32, (1, BLK), 1)
    sim = jnp.where(gk < NKEY, sim, NEG)
    sim3 = sim.reshape(NQ, NSUB_B, SUB)
    sim_ref[...] = sim3
    m_ref[...] = jnp.max(sim3, axis=2)


def _subsel_body(m_ref, out_ref, v_ref):
    v_ref[...] = m_ref[...]
    sid = lax.broadcasted_iota(jnp.int32, (NQ, NSUB), 1)
    kio = lax.broadcasted_iota(jnp.int32, (NQ, KTOP), 1)
    qid = lax.broadcasted_iota(jnp.int32, (NQ, 1), 0)

    def body(r, outv):
        v = v_ref[...]
        m = jnp.max(v, axis=1, keepdims=True)
        sel = jnp.min(jnp.where(v == m, sid, IBIG), axis=1, keepdims=True)
        v_ref[...] = jnp.where(sid == sel, NEG, v)
        return jnp.where(kio == r, qid * NSUB + sel, outv)

    out_ref[...] = lax.fori_loop(0, KTOP, body, jnp.zeros((NQ, KTOP), jnp.int32))


def _finsel_body(cand_ref, gidx_ref, out_ref, v_ref):
    v_ref[...] = cand_ref[...]
    gidx = gidx_ref[...]
    kio = lax.broadcasted_iota(jnp.int32, (NQ, KTOP), 1)

    def body(r, outv):
        v = v_ref[...]
        m = jnp.max(v, axis=1, keepdims=True)
        sel = jnp.min(jnp.where(v == m, gidx, IBIG), axis=1, keepdims=True)
        v_ref[...] = jnp.where(gidx == sel, NEG, v)
        outf = sel.astype(jnp.float32) / float(NKEY)
        return jnp.where(kio == r, outf, outv)

    out_ref[...] = lax.fori_loop(0, KTOP, body,
                                 jnp.zeros((NQ, KTOP), jnp.float32))


def _sc_gather(sim_flat, rows_flat):
    nrows = NQ * KTOP
    info = plsc.get_sparse_core_info()
    nw = info.num_cores * info.num_subcores
    per_w = nrows // nw
    mesh = plsc.VectorSubcoreMesh(core_axis_name="c", subcore_axis_name="s")

    @functools.partial(
        pl.kernel, mesh=mesh,
        out_type=jax.ShapeDtypeStruct((nrows, SUB), jnp.float32),
        compiler_params=pltpu.CompilerParams(use_tc_tiling_on_sc=True),
        scratch_types=[
            pltpu.VMEM((per_w,), jnp.int32),
            pltpu.VMEM((per_w, SUB), jnp.float32),
            pltpu.SemaphoreType.DMA,
        ],
    )
    def gather_k(sim_hbm, rows_hbm, out_hbm, idx_v, rows_v, sem):
        wid = lax.axis_index("s") * info.num_cores + lax.axis_index("c")
        base = wid * per_w
        pltpu.sync_copy(rows_hbm.at[pl.ds(base, per_w)], idx_v)
        pltpu.async_copy(sim_hbm.at[idx_v], rows_v, sem).wait()
        pltpu.sync_copy(rows_v, out_hbm.at[pl.ds(base, per_w)])

    return gather_k(sim_flat, rows_flat)


def kernel(prior_features, feature_clip, norm1_w, norm1_b, norm2_w, norm2_b):
    pf = prior_features.reshape(NKEY, DIM)
    w1 = norm1_w.reshape(1, DIM)
    b1 = norm1_b.reshape(1, DIM)
    w2 = norm2_w.reshape(1, DIM)
    b2 = norm2_b.reshape(1, DIM)

    sim3, msub = pl.pallas_call(
        _score_body,
        grid=(NBLK,),
        in_specs=[
            pl.BlockSpec((BLK, DIM), lambda b: (b, 0)),
            pl.BlockSpec((NQ, DIM), lambda b: (0, 0)),
            pl.BlockSpec((1, DIM), lambda b: (0, 0)),
            pl.BlockSpec((1, DIM), lambda b: (0, 0)),
            pl.BlockSpec((1, DIM), lambda b: (0, 0)),
            pl.BlockSpec((1, DIM), lambda b: (0, 0)),
        ],
        out_specs=[
            pl.BlockSpec((NQ, NSUB_B, SUB), lambda b: (0, b, 0)),
            pl.BlockSpec((NQ, NSUB_B), lambda b: (0, b)),
        ],
        out_shape=[
            jax.ShapeDtypeStruct((NQ, NSUB, SUB), jnp.float32),
            jax.ShapeDtypeStruct((NQ, NSUB), jnp.float32),
        ],
    )(pf, feature_clip, w1, b1, w2, b2)

    rows2 = pl.pallas_call(
        _subsel_body,
        out_shape=jax.ShapeDtypeStruct((NQ, KTOP), jnp.int32),
        scratch_shapes=[pltpu.VMEM((NQ, NSUB), jnp.float32)],
    )(msub)

    cand = _sc_gather(sim3.reshape(NQ * NSUB, SUB), rows2.reshape(-1))

    bid = rows2 - jnp.arange(NQ, dtype=jnp.int32)[:, None] * NSUB
    gidx = (bid[:, :, None] * SUB
            + jnp.arange(SUB, dtype=jnp.int32)).reshape(NQ, NCAND)

    out = pl.pallas_call(
        _finsel_body,
        out_shape=jax.ShapeDtypeStruct((NQ, KTOP), jnp.float32),
        scratch_shapes=[pltpu.VMEM((NQ, NCAND), jnp.float32)],
    )(cand.reshape(NQ, NCAND), gidx)

    return out
